# Initial kernel scaffold; baseline (speedup 1.0000x reference)
import jax
import jax.numpy as jnp
from jax.experimental import pallas as pl


def kernel(xyz1, xyz2, points1, points2):
    raise NotImplementedError("write your pallas kernel here")


# speedup vs baseline: 55.2208x; 55.2208x over previous
import jax
import jax.numpy as jnp
from jax.experimental import pallas as pl

_TI = 512
_FPAD = 64
_XCOL = 32


def _nn_kernel(f1_ref, f2_ref, idx_ref, dir_ref):
    f1 = f1_ref[0]
    f2 = f2_ref[0]
    n2 = f2.shape[0]
    dots = jax.lax.dot_general(
        f1, f2, (((1,), (1,)), ((), ())), preferred_element_type=jnp.float32
    )
    sq1 = jnp.sum(f1 * f1, axis=1, keepdims=True)
    sq2 = jnp.sum(f2 * f2, axis=1)
    d = -2.0 * dots + sq1 + sq2[None, :]
    dmin = jnp.min(d, axis=1, keepdims=True)
    jidx = jax.lax.broadcasted_iota(jnp.int32, d.shape, 1)
    idx = jnp.min(jnp.where(d == dmin, jidx, jnp.int32(n2)), axis=1)
    onehot = (jidx == idx[:, None]).astype(jnp.float32)
    xyz2 = f2[:, _XCOL:_XCOL + 8]
    nb = jax.lax.dot_general(
        onehot, xyz2, (((1,), (0,)), ((), ())), preferred_element_type=jnp.float32
    )
    dir_ref[0] = nb - f1[:, _XCOL:_XCOL + 8]
    idx_ref[0] = jnp.broadcast_to(idx[None, :], (8, _TI))


def kernel(xyz1, xyz2, points1, points2):
    B, C, N1 = xyz1.shape
    N2 = xyz2.shape[2]
    D = points1.shape[1]
    F = D + C
    f1 = jnp.transpose(jnp.concatenate([points1, xyz1], axis=1), (0, 2, 1))
    f2 = jnp.transpose(jnp.concatenate([points2, xyz2], axis=1), (0, 2, 1))
    f1 = jnp.pad(f1, ((0, 0), (0, 0), (0, _FPAD - F)))
    f2 = jnp.pad(f2, ((0, 0), (0, 0), (0, _FPAD - F)))

    idx_out, dir_out = pl.pallas_call(
        _nn_kernel,
        grid=(B, N1 // _TI),
        in_specs=[
            pl.BlockSpec((1, _TI, _FPAD), lambda b, i: (b, i, 0)),
            pl.BlockSpec((1, N2, _FPAD), lambda b, i: (b, 0, 0)),
        ],
        out_specs=[
            pl.BlockSpec((1, 8, _TI), lambda b, i: (b, 0, i)),
            pl.BlockSpec((1, _TI, 8), lambda b, i: (b, i, 0)),
        ],
        out_shape=[
            jax.ShapeDtypeStruct((B, 8, N1), jnp.int32),
            jax.ShapeDtypeStruct((B, N1, 8), jnp.float32),
        ],
    )(f1, f2)

    corres2 = idx_out[:, :1, :]
    direction = jnp.transpose(dir_out[:, :, :3], (0, 2, 1))
    corres1 = jnp.broadcast_to(
        jnp.arange(N1, dtype=jnp.int32)[None, None, :], (B, 1, N1)
    )
    return (corres1, corres2, direction)
